# Initial kernel scaffold; baseline (speedup 1.0000x reference)
import jax
import jax.numpy as jnp
from jax.experimental import pallas as pl


def kernel(inputs, table, W1, b1, W2, b2):
    raise NotImplementedError("write your pallas kernel here")


# speedup vs baseline: 1.8785x; 1.8785x over previous
import functools

import jax
import jax.numpy as jnp
from jax import lax
from jax.experimental import pallas as pl
from jax.experimental.pallas import tpu as pltpu
from jax.experimental.pallas import tpu_sc as plsc

VOCAB = 1000000
EMBED = 32
BATCH = 4096
SEQ = 200

NC = 2
NS = 16
NW = NC * NS
B_PER_W = BATCH // NW
CHUNK = 128
TOTAL = BATCH * SEQ
N_CHUNKS = TOTAL // (NW * CHUNK)


def _sc_pooled_sum(idx2d, seg2d, table):
    mesh = plsc.VectorSubcoreMesh(core_axis_name="c", subcore_axis_name="s")

    @functools.partial(
        pl.kernel,
        out_type=jax.ShapeDtypeStruct((BATCH, EMBED), jnp.float32),
        mesh=mesh,
        scratch_types=[
            pltpu.VMEM((N_CHUNKS, CHUNK), jnp.int32),
            pltpu.VMEM((N_CHUNKS, CHUNK), jnp.int32),
            pltpu.VMEM((B_PER_W, EMBED), jnp.float32),
            pltpu.VMEM_SHARED((BATCH, EMBED), jnp.float32),
        ],
        compiler_params=pltpu.CompilerParams(use_tc_tiling_on_sc=False),
    )
    def sc_kernel(idx_hbm, seg_hbm, table_hbm, out_hbm, idx_v, seg_v, rows_v,
                  pooled_sh):
        wid = lax.axis_index("s") * NC + lax.axis_index("c")
        row0 = wid * B_PER_W
        c0 = wid * N_CHUNKS

        pltpu.sync_copy(idx_hbm.at[pl.ds(c0, N_CHUNKS)], idx_v)
        pltpu.sync_copy(seg_hbm.at[pl.ds(c0, N_CHUNKS)], seg_v)

        z = jnp.zeros((16,), jnp.float32)

        @pl.loop(0, B_PER_W)
        def _(i):
            rows_v[i, pl.ds(0, 16)] = z
            rows_v[i, pl.ds(16, 16)] = z

        pltpu.sync_copy(rows_v, pooled_sh.at[pl.ds(row0, B_PER_W)])

        @pl.loop(0, N_CHUNKS)
        def _(j):
            pltpu.sync_copy(table_hbm.at[idx_v.at[j]], rows_v)
            pltpu.sync_copy(rows_v, pooled_sh.at[seg_v.at[j]], add=True)

        pltpu.sync_copy(pooled_sh.at[pl.ds(row0, B_PER_W)],
                        out_hbm.at[pl.ds(row0, B_PER_W)])

    return sc_kernel(idx2d, seg2d, table)


def _dense_head_body(x_ref, w1_ref, b1_ref, w2_ref, b2_ref, o_ref):
    x = x_ref[...] * jnp.float32(1.0 / SEQ)
    h = jnp.dot(x, w1_ref[...], preferred_element_type=jnp.float32)
    h = jnp.maximum(h + b1_ref[...], 0.0)
    logits = jnp.dot(h, w2_ref[...], preferred_element_type=jnp.float32)
    logits = logits + b2_ref[...]
    m = jnp.max(logits, axis=-1, keepdims=True)
    e = jnp.exp(logits - m)
    o_ref[...] = e / jnp.sum(e, axis=-1, keepdims=True)


def kernel(inputs, table, W1, b1, W2, b2):
    idx2d = inputs.astype(jnp.int32).reshape(TOTAL // CHUNK, CHUNK)
    seg2d = (lax.iota(jnp.int32, TOTAL) // SEQ).reshape(TOTAL // CHUNK, CHUNK)

    pooled = _sc_pooled_sum(idx2d, seg2d, table)

    out = pl.pallas_call(
        _dense_head_body,
        out_shape=jax.ShapeDtypeStruct((BATCH, 4), jnp.float32),
    )(pooled, W1, b1.reshape(1, 64), W2, b2.reshape(1, 4))
    return out


# speedup vs baseline: 2.1904x; 1.1660x over previous
import functools

import jax
import jax.numpy as jnp
from jax import lax
from jax.experimental import pallas as pl
from jax.experimental.pallas import tpu as pltpu
from jax.experimental.pallas import tpu_sc as plsc

VOCAB = 1000000
EMBED = 32
BATCH = 4096
SEQ = 200

NC = 2
NS = 16
NW = NC * NS
B_PER_W = BATCH // NW
CHUNK = 128
TOTAL = BATCH * SEQ
N_CHUNKS = TOTAL // (NW * CHUNK)
K = 4
N_GROUPS = N_CHUNKS // K


def _sc_pooled_sum(idx2d, seg2d, table):
    mesh = plsc.VectorSubcoreMesh(core_axis_name="c", subcore_axis_name="s")

    @functools.partial(
        pl.kernel,
        out_type=jax.ShapeDtypeStruct((BATCH, EMBED), jnp.float32),
        mesh=mesh,
        scratch_types=[
            pltpu.VMEM((N_CHUNKS, CHUNK), jnp.int32),
            pltpu.VMEM((N_CHUNKS, CHUNK), jnp.int32),
            pltpu.VMEM((K * CHUNK, EMBED), jnp.float32),
            pltpu.VMEM((K * CHUNK, EMBED), jnp.float32),
            pltpu.VMEM_SHARED((BATCH, EMBED), jnp.float32),
            pltpu.SemaphoreType.DMA,
            pltpu.SemaphoreType.DMA,
        ],
        compiler_params=pltpu.CompilerParams(use_tc_tiling_on_sc=False),
    )
    def sc_kernel(idx_hbm, seg_hbm, table_hbm, out_hbm, idx_v, seg_v,
                  rows_a, rows_b, pooled_sh, sem_a, sem_b):
        wid = lax.axis_index("s") * NC + lax.axis_index("c")
        row0 = wid * B_PER_W
        c0 = wid * N_CHUNKS

        pltpu.sync_copy(idx_hbm.at[pl.ds(c0, N_CHUNKS)], idx_v)
        pltpu.sync_copy(seg_hbm.at[pl.ds(c0, N_CHUNKS)], seg_v)

        z = jnp.zeros((16,), jnp.float32)

        @pl.loop(0, B_PER_W)
        def _(i):
            rows_a[i, pl.ds(0, 16)] = z
            rows_a[i, pl.ds(16, 16)] = z

        pltpu.sync_copy(rows_a.at[pl.ds(0, B_PER_W)],
                        pooled_sh.at[pl.ds(row0, B_PER_W)])

        def fire(g, rows, sem):
            for u in range(K):
                pltpu.async_copy(table_hbm.at[idx_v.at[g * K + u]],
                                 rows.at[pl.ds(u * CHUNK, CHUNK)], sem)

        def drain(g, rows, sem):
            for u in range(K):
                pltpu.make_async_copy(table_hbm.at[idx_v.at[g * K + u]],
                                      rows.at[pl.ds(u * CHUNK, CHUNK)],
                                      sem).wait()

        def scat(g, rows):
            for u in range(K):
                pltpu.sync_copy(rows.at[pl.ds(u * CHUNK, CHUNK)],
                                pooled_sh.at[seg_v.at[g * K + u]], add=True)

        fire(0, rows_a, sem_a)

        @pl.loop(0, N_GROUPS // 2)
        def _(t):
            g = t * 2
            drain(g, rows_a, sem_a)
            fire(g + 1, rows_b, sem_b)
            scat(g, rows_a)
            drain(g + 1, rows_b, sem_b)

            @pl.when(g + 2 < N_GROUPS)
            def _():
                fire(g + 2, rows_a, sem_a)

            scat(g + 1, rows_b)

        pltpu.sync_copy(pooled_sh.at[pl.ds(row0, B_PER_W)],
                        out_hbm.at[pl.ds(row0, B_PER_W)])

    return sc_kernel(idx2d, seg2d, table)


def _dense_head_body(x_ref, w1_ref, b1_ref, w2_ref, b2_ref, o_ref):
    x = x_ref[...] * jnp.float32(1.0 / SEQ)
    h = jnp.dot(x, w1_ref[...], preferred_element_type=jnp.float32)
    h = jnp.maximum(h + b1_ref[...], 0.0)
    logits = jnp.dot(h, w2_ref[...], preferred_element_type=jnp.float32)
    logits = logits + b2_ref[...]
    m = jnp.max(logits, axis=-1, keepdims=True)
    e = jnp.exp(logits - m)
    o_ref[...] = e / jnp.sum(e, axis=-1, keepdims=True)


def kernel(inputs, table, W1, b1, W2, b2):
    idx2d = inputs.astype(jnp.int32).reshape(TOTAL // CHUNK, CHUNK)
    seg2d = (lax.iota(jnp.int32, TOTAL) // SEQ).reshape(TOTAL // CHUNK, CHUNK)

    pooled = _sc_pooled_sum(idx2d, seg2d, table)

    out = pl.pallas_call(
        _dense_head_body,
        out_shape=jax.ShapeDtypeStruct((BATCH, 4), jnp.float32),
    )(pooled, W1, b1.reshape(1, 64), W2, b2.reshape(1, 4))
    return out


# speedup vs baseline: 2.2040x; 1.0062x over previous
import functools

import jax
import jax.numpy as jnp
from jax import lax
from jax.experimental import pallas as pl
from jax.experimental.pallas import tpu as pltpu
from jax.experimental.pallas import tpu_sc as plsc

VOCAB = 1000000
EMBED = 32
BATCH = 4096
SEQ = 200

NC = 2
NS = 16
NW = NC * NS
B_PER_W = BATCH // NW
CHUNK = 128
TOTAL = BATCH * SEQ
N_CHUNKS = TOTAL // (NW * CHUNK)
K = 4
N_GROUPS = N_CHUNKS // K
NSUB = 4


def _sc_pooled_sum(idx2d, seg2d, table):
    mesh = plsc.VectorSubcoreMesh(core_axis_name="c", subcore_axis_name="s")

    @functools.partial(
        pl.kernel,
        out_type=jax.ShapeDtypeStruct((BATCH, EMBED), jnp.float32),
        mesh=mesh,
        scratch_types=[
            pltpu.VMEM((N_CHUNKS, CHUNK), jnp.int32),
            pltpu.VMEM((N_CHUNKS, CHUNK), jnp.int32),
            pltpu.VMEM((K * CHUNK, EMBED), jnp.float32),
            pltpu.VMEM((K * CHUNK, EMBED), jnp.float32),
            pltpu.VMEM_SHARED((NSUB * BATCH, EMBED), jnp.float32),
            pltpu.SemaphoreType.DMA,
            pltpu.SemaphoreType.DMA,
        ],
        compiler_params=pltpu.CompilerParams(use_tc_tiling_on_sc=False),
    )
    def sc_kernel(idx_hbm, seg_hbm, table_hbm, out_hbm, idx_v, seg_v,
                  rows_a, rows_b, acc_sh, sem_a, sem_b):
        wid = lax.axis_index("s") * NC + lax.axis_index("c")
        row0 = wid * B_PER_W
        arow0 = NSUB * row0
        c0 = wid * N_CHUNKS

        pltpu.sync_copy(idx_hbm.at[pl.ds(c0, N_CHUNKS)], idx_v)
        pltpu.sync_copy(seg_hbm.at[pl.ds(c0, N_CHUNKS)], seg_v)

        z = jnp.zeros((16,), jnp.float32)

        @pl.loop(0, NSUB * B_PER_W)
        def _(i):
            rows_a[i, pl.ds(0, 16)] = z
            rows_a[i, pl.ds(16, 16)] = z

        pltpu.sync_copy(rows_a, acc_sh.at[pl.ds(arow0, NSUB * B_PER_W)])

        def fire(g, rows, sem):
            for u in range(K):
                pltpu.async_copy(table_hbm.at[idx_v.at[g * K + u]],
                                 rows.at[pl.ds(u * CHUNK, CHUNK)], sem)

        def drain(g, rows, sem):
            for u in range(K):
                pltpu.make_async_copy(table_hbm.at[idx_v.at[g * K + u]],
                                      rows.at[pl.ds(u * CHUNK, CHUNK)],
                                      sem).wait()

        def scat(g, rows):
            for u in range(K):
                pltpu.sync_copy(rows.at[pl.ds(u * CHUNK, CHUNK)],
                                acc_sh.at[seg_v.at[g * K + u]], add=True)

        fire(0, rows_a, sem_a)

        @pl.loop(0, N_GROUPS // 2)
        def _(t):
            g = t * 2
            drain(g, rows_a, sem_a)
            fire(g + 1, rows_b, sem_b)
            scat(g, rows_a)
            drain(g + 1, rows_b, sem_b)

            @pl.when(g + 2 < N_GROUPS)
            def _():
                fire(g + 2, rows_a, sem_a)

            scat(g + 1, rows_b)

        pltpu.sync_copy(acc_sh.at[pl.ds(arow0, NSUB * B_PER_W)], rows_a)

        @pl.loop(0, B_PER_W)
        def _(i):
            r = NSUB * i
            rows_b[i, pl.ds(0, 16)] = (
                (rows_a[r, pl.ds(0, 16)] + rows_a[r + 1, pl.ds(0, 16)])
                + (rows_a[r + 2, pl.ds(0, 16)] + rows_a[r + 3, pl.ds(0, 16)]))
            rows_b[i, pl.ds(16, 16)] = (
                (rows_a[r, pl.ds(16, 16)] + rows_a[r + 1, pl.ds(16, 16)])
                + (rows_a[r + 2, pl.ds(16, 16)] + rows_a[r + 3, pl.ds(16, 16)]))

        pltpu.sync_copy(rows_b.at[pl.ds(0, B_PER_W)],
                        out_hbm.at[pl.ds(row0, B_PER_W)])

    return sc_kernel(idx2d, seg2d, table)


def _dense_head_body(x_ref, w1_ref, b1_ref, w2_ref, b2_ref, o_ref):
    x = x_ref[...] * jnp.float32(1.0 / SEQ)
    h = jnp.dot(x, w1_ref[...], preferred_element_type=jnp.float32)
    h = jnp.maximum(h + b1_ref[...], 0.0)
    logits = jnp.dot(h, w2_ref[...], preferred_element_type=jnp.float32)
    logits = logits + b2_ref[...]
    m = jnp.max(logits, axis=-1, keepdims=True)
    e = jnp.exp(logits - m)
    o_ref[...] = e / jnp.sum(e, axis=-1, keepdims=True)


def kernel(inputs, table, W1, b1, W2, b2):
    idx2d = inputs.astype(jnp.int32).reshape(TOTAL // CHUNK, CHUNK)
    flat = lax.iota(jnp.int32, TOTAL)
    seg2d = ((flat // SEQ) * NSUB
             + (flat // CHUNK) % NSUB).reshape(TOTAL // CHUNK, CHUNK)

    pooled = _sc_pooled_sum(idx2d, seg2d, table)

    out = pl.pallas_call(
        _dense_head_body,
        out_shape=jax.ShapeDtypeStruct((BATCH, 4), jnp.float32),
    )(pooled, W1, b1.reshape(1, 64), W2, b2.reshape(1, 4))
    return out
